# baseline (speedup 1.0000x reference)
import functools

import jax
import jax.numpy as jnp
from jax import lax
from jax.experimental import pallas as pl
from jax.experimental.pallas import tpu as pltpu
from jax.experimental.pallas import tpu_sc as plsc

_INFO = plsc.get_sparse_core_info()
_NC = _INFO.num_cores
_NS = _INFO.num_subcores
_NW = _NC * _NS

_N, _C, _H, _W = 16, 512, 64, 64
_HALF = _C // 2
_SITES_PER_W = (_N * _H) // _NW
_NBUF = 4

_mesh = plsc.VectorSubcoreMesh(core_axis_name="c", subcore_axis_name="s")


@functools.partial(
    pl.kernel,
    out_type=jax.ShapeDtypeStruct((_N, _H, _W, _C), jnp.float32),
    mesh=_mesh,
    compiler_params=pltpu.CompilerParams(use_tc_tiling_on_sc=True),
    scratch_types=(
        [pltpu.VMEM_SHARED((_NS, _NBUF - 1, _W, _C), jnp.float32)]
        + [pltpu.VMEM((_W, _C), jnp.float32)]
        + [pltpu.SemaphoreType.DMA] * (2 * _NBUF)
    ),
)
def _flip_copy(x_hbm, out_hbm, spmem, tbuf, *sems):
    sid = lax.axis_index("s")
    wid = sid * _NC + lax.axis_index("c")
    n = wid // 2
    h0 = (wid % 2) * _SITES_PER_W

    bufs = tuple(spmem.at[sid, b] for b in range(_NBUF - 1)) + (tbuf,)
    in_sems = sems[:_NBUF]
    out_sems = sems[_NBUF:]
    in_cp = [None] * _NBUF
    out_cp = [[] for _ in range(_NBUF)]

    def start_fetch(i):
        b = i % _NBUF
        for cp in out_cp[b]:
            cp.wait()
        out_cp[b] = []
        in_cp[b] = pltpu.async_copy(x_hbm.at[n, h0 + i], bufs[b], in_sems[b])

    for i in range(min(_NBUF, _SITES_PER_W)):
        start_fetch(i)
    for i in range(_SITES_PER_W):
        b = i % _NBUF
        in_cp[b].wait()
        h = h0 + i
        out_cp[b] = [
            pltpu.async_copy(
                bufs[b].at[:, pl.ds(_HALF, _HALF)],
                out_hbm.at[n, h, :, pl.ds(0, _HALF)],
                out_sems[b],
            ),
            pltpu.async_copy(
                bufs[b].at[:, pl.ds(0, _HALF)],
                out_hbm.at[n, h, :, pl.ds(_HALF, _HALF)],
                out_sems[b],
            ),
        ]
        nxt = i + _NBUF
        if nxt < _SITES_PER_W:
            start_fetch(nxt)

    for b in range(_NBUF):
        for cp in out_cp[b]:
            cp.wait()


def kernel(x):
    x_t = jnp.transpose(x, (0, 2, 3, 1))
    y_t = _flip_copy(x_t)
    return jnp.transpose(y_t, (0, 3, 1, 2))


# speedup vs baseline: 1.0444x; 1.0444x over previous
import functools

import jax
import jax.numpy as jnp
from jax import lax
from jax.experimental import pallas as pl
from jax.experimental.pallas import tpu as pltpu
from jax.experimental.pallas import tpu_sc as plsc

_INFO = plsc.get_sparse_core_info()
_NC = _INFO.num_cores
_NS = _INFO.num_subcores
_NW = _NC * _NS

_N, _C, _H, _W = 16, 512, 64, 64
_HALF = _C // 2
_SITES_PER_W = (_N * _H) // _NW
_NBUF = 3

_mesh = plsc.VectorSubcoreMesh(core_axis_name="c", subcore_axis_name="s")


@functools.partial(
    pl.kernel,
    out_type=jax.ShapeDtypeStruct((_N, _H, _W, _C), jnp.float32),
    mesh=_mesh,
    compiler_params=pltpu.CompilerParams(use_tc_tiling_on_sc=True),
    scratch_types=(
        [pltpu.VMEM_SHARED((_NS, _NBUF, _W, _C), jnp.float32)]
        + [pltpu.SemaphoreType.DMA] * (2 * _NBUF)
    ),
)
def _flip_copy(x_hbm, out_hbm, spmem, *sems):
    sid = lax.axis_index("s")
    wid = sid * _NC + lax.axis_index("c")
    n = wid // 2
    h0 = (wid % 2) * _SITES_PER_W

    bufs = tuple(spmem.at[sid, b] for b in range(_NBUF))
    in_sems = sems[:_NBUF]
    out_sems = sems[_NBUF:]
    in_cp = [None] * _NBUF
    out_cp = [[] for _ in range(_NBUF)]

    def start_fetch(i):
        b = i % _NBUF
        for cp in out_cp[b]:
            cp.wait()
        out_cp[b] = []
        in_cp[b] = pltpu.async_copy(x_hbm.at[n, h0 + i], bufs[b], in_sems[b])

    for i in range(min(_NBUF, _SITES_PER_W)):
        start_fetch(i)
    for i in range(_SITES_PER_W):
        b = i % _NBUF
        in_cp[b].wait()
        h = h0 + i
        out_cp[b] = [
            pltpu.async_copy(
                bufs[b].at[:, pl.ds(_HALF, _HALF)],
                out_hbm.at[n, h, :, pl.ds(0, _HALF)],
                out_sems[b],
            ),
            pltpu.async_copy(
                bufs[b].at[:, pl.ds(0, _HALF)],
                out_hbm.at[n, h, :, pl.ds(_HALF, _HALF)],
                out_sems[b],
            ),
        ]
        nxt = i + _NBUF
        if nxt < _SITES_PER_W:
            start_fetch(nxt)

    for b in range(_NBUF):
        for cp in out_cp[b]:
            cp.wait()


def kernel(x):
    x_t = jnp.transpose(x, (0, 2, 3, 1))
    y_t = _flip_copy(x_t)
    return jnp.transpose(y_t, (0, 3, 1, 2))
